# Initial kernel scaffold; baseline (speedup 1.0000x reference)
import jax
import jax.numpy as jnp
from jax.experimental import pallas as pl


def kernel(x, edge_index, batch, W1, att_src1, att_dst1, bias1, W2, att_src2, att_dst2, bias2, lin1_W, lin1_b, lin2_W, lin2_b):
    raise NotImplementedError("write your pallas kernel here")


# speedup vs baseline: 1.1674x; 1.1674x over previous
import jax
import jax.numpy as jnp
from jax.experimental import pallas as pl


N_NODES = 10000
N_GRAPHS = 64


def _mm_kernel(x_ref, w_ref, o_ref):
    o_ref[...] = jnp.dot(x_ref[...], w_ref[...], preferred_element_type=jnp.float32)


def _matmul(x, w):
    m, k = x.shape
    _, n = w.shape
    bm = 2000
    return pl.pallas_call(
        _mm_kernel,
        grid=(m // bm,),
        in_specs=[pl.BlockSpec((bm, k), lambda i: (i, 0)),
                  pl.BlockSpec((k, n), lambda i: (0, 0))],
        out_specs=pl.BlockSpec((bm, n), lambda i: (i, 0)),
        out_shape=jax.ShapeDtypeStruct((m, n), jnp.float32),
    )(x, w)


def _gat_fused(x, src, dst, W, att_src, att_dst, bias, heads, out_ch):
    N = x.shape[0]
    h = _matmul(x, W).reshape(N, heads, out_ch)
    a_src = (h * att_src[None]).sum(-1)
    a_dst = (h * att_dst[None]).sum(-1)
    e = a_src[src] + a_dst[dst]
    e = jnp.where(e > 0, e, 0.2 * e)
    w = jnp.exp(e)
    s = jax.ops.segment_sum(w, dst, num_segments=N)
    acc = jax.ops.segment_sum(h[src] * w[:, :, None], dst, num_segments=N)
    out = acc / (s[:, :, None] + 1e-16)
    return out.reshape(N, heads * out_ch) + bias


def kernel(x, edge_index, batch, W1, att_src1, att_dst1, bias1, W2, att_src2,
           att_dst2, bias2, lin1_W, lin1_b, lin2_W, lin2_b):
    N = x.shape[0]
    sl = jnp.arange(N, dtype=edge_index.dtype)
    src = jnp.concatenate([edge_index[0], sl])
    dst = jnp.concatenate([edge_index[1], sl])
    h = jax.nn.elu(_gat_fused(x, src, dst, W1, att_src1, att_dst1, bias1, 8, 64))
    h = jax.nn.elu(_gat_fused(h, src, dst, W2, att_src2, att_dst2, bias2, 1, 64))
    g = jax.ops.segment_sum(h, batch, num_segments=N_GRAPHS)
    g = jax.nn.elu(g @ lin1_W + lin1_b)
    return jax.nn.log_softmax(g @ lin2_W + lin2_b, axis=-1)


# speedup vs baseline: 10.6971x; 9.1632x over previous
import functools

import jax
import jax.numpy as jnp
from jax import lax
from jax.experimental import pallas as pl
from jax.experimental.pallas import tpu as pltpu
from jax.experimental.pallas import tpu_sc as plsc

N = 10000
NP = 10016
E_REAL = 330000
E_PAD = 344064
HEADS = 8
HID = 64
NG = 64
F32 = jnp.float32

NC, NS = 2, 16
NW = NC * NS


def _mesh():
    return plsc.VectorSubcoreMesh(core_axis_name="c", subcore_axis_name="s",
                                  num_cores=NC, num_subcores=NS)


def _tc_in_body(x_ref, w1_ref, ws1_ref, h1t_ref, a1t_ref):
    x = x_ref[...]
    h1t_ref[...] = lax.dot_general(w1_ref[...], x, (((0,), (1,)), ((), ())),
                                   preferred_element_type=F32)
    a1t_ref[...] = lax.dot_general(ws1_ref[...], x, (((0,), (1,)), ((), ())),
                                   preferred_element_type=F32)


def _make_attn_body(nheads, chunk):
    split = NW // nheads
    erange = E_PAD // split
    nchunks = erange // chunk
    ngroups = chunk // 16

    def body(a_hbm, src_hbm, dst_hbm, w_out, s_out, asrc_v, adst_v, s_v,
             src_v, dst_v, w_v):
        wid = lax.axis_index("s") * NC + lax.axis_index("c")
        hd = wid // split
        q = wid % split
        pltpu.sync_copy(a_hbm.at[hd], asrc_v.at[pl.ds(0, N)])
        pltpu.sync_copy(a_hbm.at[nheads + hd], adst_v.at[pl.ds(0, N)])
        adst_v[pl.ds(N, 16)] = jnp.full((16,), -30.0, F32)

        def zero(i, c):
            s_v[pl.ds(i * 16, 16)] = jnp.zeros((16,), F32)
            return c
        lax.fori_loop(0, NP // 16, zero, 0)

        base0 = q * erange

        def do_chunk(ci, c):
            b = base0 + ci * chunk
            pltpu.sync_copy(src_hbm.at[pl.ds(b, chunk)], src_v)
            pltpu.sync_copy(dst_hbm.at[pl.ds(b, chunk)], dst_v)

            def grp(g, cc):
                sv = src_v[pl.ds(g * 16, 16)]
                dv = dst_v[pl.ds(g * 16, 16)]
                e = plsc.load_gather(asrc_v, [sv]) + plsc.load_gather(adst_v, [dv])
                e = jnp.where(e > 0, e, 0.2 * e)
                w = jnp.exp(e)
                w_v[pl.ds(g * 16, 16)] = w
                plsc.addupdate_scatter(s_v, [dv], w)
                return cc
            lax.fori_loop(0, ngroups, grp, 0)
            pltpu.sync_copy(w_v, w_out.at[hd, pl.ds(b, chunk)])
            return c
        lax.fori_loop(0, nchunks, do_chunk, 0)
        pltpu.sync_copy(s_v, s_out.at[wid])

    return body


def _attn_pass(a1t, srcp, dstp, nheads, chunk):
    body = _make_attn_body(nheads, chunk)
    f = pl.kernel(
        body,
        out_type=(jax.ShapeDtypeStruct((nheads, E_PAD), F32),
                  jax.ShapeDtypeStruct((NW, NP), F32)),
        mesh=_mesh(),
        scratch_types=[
            pltpu.VMEM((NP,), F32), pltpu.VMEM((NP,), F32), pltpu.VMEM((NP,), F32),
            pltpu.VMEM((chunk,), jnp.int32), pltpu.VMEM((chunk,), jnp.int32),
            pltpu.VMEM((chunk,), F32),
        ],
        compiler_params=pltpu.CompilerParams(needs_layout_passes=False, use_tc_tiling_on_sc=False),
    )
    return f(a1t, srcp, dstp)


def _make_msg_body(nch, npass, nheads, chunk):
    ngroups = chunk // 16
    nchunks = E_PAD // chunk

    def body(tab_hbm, src_hbm, dst_hbm, w_hbm, out_t, *refs):
        tabs = refs[:nch]
        accs = refs[nch:2 * nch]
        src_v, dst_v, w_v = refs[2 * nch:2 * nch + 3]
        wid = lax.axis_index("s") * NC + lax.axis_index("c")

        def do_pass(p, c):
            ch0 = p * (NW * nch) + wid * nch
            hd = ch0 // HID if nheads > 1 else 0
            for k in range(nch):
                pltpu.sync_copy(tab_hbm.at[ch0 + k], tabs[k].at[pl.ds(0, N)])

            def zero(i, cc):
                for k in range(nch):
                    accs[k][pl.ds(i * 16, 16)] = jnp.zeros((16,), F32)
                return cc
            lax.fori_loop(0, NP // 16, zero, 0)

            def do_chunk(ci, cc):
                b = ci * chunk
                pltpu.sync_copy(src_hbm.at[pl.ds(b, chunk)], src_v)
                pltpu.sync_copy(dst_hbm.at[pl.ds(b, chunk)], dst_v)
                pltpu.sync_copy(w_hbm.at[hd, pl.ds(b, chunk)], w_v)

                def grp(g, ccc):
                    sv = src_v[pl.ds(g * 16, 16)]
                    dv = dst_v[pl.ds(g * 16, 16)]
                    wv = w_v[pl.ds(g * 16, 16)]
                    for k in range(nch):
                        rows = plsc.load_gather(tabs[k], [sv])
                        plsc.addupdate_scatter(accs[k], [dv], rows * wv)
                    return ccc
                lax.fori_loop(0, ngroups, grp, 0)
                return cc
            lax.fori_loop(0, nchunks, do_chunk, 0)

            for k in range(nch):
                pltpu.sync_copy(accs[k].at[pl.ds(0, N)], out_t.at[ch0 + k])
            return c
        lax.fori_loop(0, npass, do_pass, 0)

    return body


def _msg_pass(tab, srcp, dstp, w_e, nch, npass, nheads, chunk):
    nchan = tab.shape[0]
    body = _make_msg_body(nch, npass, nheads, chunk)
    scr = ([pltpu.VMEM((NP,), F32)] * (2 * nch) +
           [pltpu.VMEM((chunk,), jnp.int32), pltpu.VMEM((chunk,), jnp.int32),
            pltpu.VMEM((chunk,), F32)])
    f = pl.kernel(
        body,
        out_type=jax.ShapeDtypeStruct((nchan, N), F32),
        mesh=_mesh(),
        scratch_types=scr,
        compiler_params=pltpu.CompilerParams(needs_layout_passes=False, use_tc_tiling_on_sc=False),
    )
    return f(tab, srcp, dstp, w_e)


def _norm_body(acc_ref, sp_ref, bias_ref, out_ref):
    s = jnp.sum(sp_ref[...], axis=1)[:, :N]
    v = acc_ref[...] / (s + 1e-16) + bias_ref[...]
    out_ref[...] = jnp.where(v > 0, v, jnp.exp(v) - 1.0)


def _l2in_body(h1n_ref, w2_ref, ws2_ref, h2t_ref, a2t_ref):
    h1n = h1n_ref[...]
    h2t_ref[...] = lax.dot_general(w2_ref[...], h1n, (((0,), (0,)), ((), ())),
                                   preferred_element_type=F32)
    a2t_ref[...] = lax.dot_general(ws2_ref[...], h1n, (((0,), (0,)), ((), ())),
                                   preferred_element_type=F32)


def _final_body(acc2_ref, s2p_ref, bias2_ref, batch_ref, l1w_ref, l1b_ref,
                l2w_ref, l2b_ref, out_ref):
    s2 = jnp.sum(s2p_ref[...], axis=0, keepdims=True)[:, :N]
    v = acc2_ref[...] / (s2 + 1e-16) + bias2_ref[...]
    h = jnp.where(v > 0, v, jnp.exp(v) - 1.0)
    gid = lax.broadcasted_iota(jnp.int32, (N, NG), 1)
    P = (batch_ref[...] == gid).astype(F32)
    G = lax.dot_general(P, h, (((0,), (1,)), ((), ())),
                        preferred_element_type=F32)
    g1 = jnp.dot(G, l1w_ref[...], preferred_element_type=F32) + l1b_ref[...]
    g1 = jnp.where(g1 > 0, g1, jnp.exp(g1) - 1.0)
    lg = jnp.dot(g1, l2w_ref[...], preferred_element_type=F32) + l2b_ref[...]
    m = jnp.max(lg, axis=1, keepdims=True)
    lse = jnp.log(jnp.sum(jnp.exp(lg - m), axis=1, keepdims=True)) + m
    out_ref[...] = lg - lse


def kernel(x, edge_index, batch, W1, att_src1, att_dst1, bias1, W2, att_src2,
           att_dst2, bias2, lin1_W, lin1_b, lin2_W, lin2_b):
    sl = jnp.arange(N, dtype=jnp.int32)
    npad = E_PAD - E_REAL
    srcp = jnp.concatenate([edge_index[0].astype(jnp.int32), sl,
                            jnp.zeros((npad,), jnp.int32)])
    dstp = jnp.concatenate([edge_index[1].astype(jnp.int32), sl,
                            jnp.full((npad,), N, jnp.int32)])

    w1r = W1.reshape(x.shape[1], HEADS, HID)
    ws1 = jnp.concatenate([jnp.einsum('khc,hc->kh', w1r, att_src1),
                           jnp.einsum('khc,hc->kh', w1r, att_dst1)], axis=1)
    ws2 = jnp.stack([W2 @ att_src2[0], W2 @ att_dst2[0]], axis=1)

    h1t, a1t = pl.pallas_call(
        _tc_in_body,
        out_shape=(jax.ShapeDtypeStruct((HEADS * HID, N), F32),
                   jax.ShapeDtypeStruct((2 * HEADS, N), F32)),
    )(x, W1, ws1)

    w1e, s1p = _attn_pass(a1t, srcp, dstp, HEADS, 2048)
    out1t = _msg_pass(h1t, srcp, dstp, w1e, nch=4, npass=4, nheads=HEADS,
                      chunk=2048)

    h1n = pl.pallas_call(
        _norm_body,
        grid=(HEADS,),
        in_specs=[pl.BlockSpec((HID, N), lambda h: (h, 0)),
                  pl.BlockSpec((1, NW // HEADS, NP), lambda h: (h, 0, 0)),
                  pl.BlockSpec((HID, 1), lambda h: (h, 0))],
        out_specs=pl.BlockSpec((HID, N), lambda h: (h, 0)),
        out_shape=jax.ShapeDtypeStruct((HEADS * HID, N), F32),
    )(out1t, s1p.reshape(HEADS, NW // HEADS, NP), bias1.reshape(HEADS * HID, 1))

    h2t, a2t = pl.pallas_call(
        _l2in_body,
        out_shape=(jax.ShapeDtypeStruct((HID, N), F32),
                   jax.ShapeDtypeStruct((2, N), F32)),
    )(h1n, W2, ws2)

    w2e, s2p = _attn_pass(a2t, srcp, dstp, 1, 1344)
    out2t = _msg_pass(h2t, srcp, dstp, w2e, nch=2, npass=1, nheads=1,
                      chunk=2048)

    out = pl.pallas_call(
        _final_body,
        out_shape=jax.ShapeDtypeStruct((NG, 16), F32),
    )(out2t, s2p, bias2.reshape(HID, 1), batch.reshape(N, 1).astype(jnp.int32),
      lin1_W, lin1_b.reshape(1, HID), lin2_W, lin2_b.reshape(1, 16))
    return out


# speedup vs baseline: 15.6766x; 1.4655x over previous
import functools

import jax
import jax.numpy as jnp
from jax import lax
from jax.experimental import pallas as pl
from jax.experimental.pallas import tpu as pltpu
from jax.experimental.pallas import tpu_sc as plsc

N = 10000
NP = 10016
E_REAL = 330000
E_PAD = 344064
HEADS = 8
HID = 64
NG = 64
F32 = jnp.float32

NC, NS = 2, 16
NW = NC * NS


def _mesh():
    return plsc.VectorSubcoreMesh(core_axis_name="c", subcore_axis_name="s",
                                  num_cores=NC, num_subcores=NS)


def _tc_in_body(x_ref, w1_ref, ws1_ref, h1t_ref, a1t_ref):
    x = x_ref[...]
    h1t_ref[...] = lax.dot_general(w1_ref[...], x, (((0,), (1,)), ((), ())),
                                   preferred_element_type=F32)
    a1t_ref[...] = lax.dot_general(ws1_ref[...], x, (((0,), (1,)), ((), ())),
                                   preferred_element_type=F32)


def _make_attn_body(nheads, chunk):
    split = NW // nheads
    erange = E_PAD // split
    nchunks = erange // chunk
    ngroups = chunk // 16

    def body(a_hbm, src_hbm, dst_hbm, w_out, s_out, asrc_v, adst_v, s_v,
             src_v, dst_v, w_v):
        wid = lax.axis_index("s") * NC + lax.axis_index("c")
        hd = wid // split
        q = wid % split
        pltpu.sync_copy(a_hbm.at[hd], asrc_v.at[pl.ds(0, N)])
        pltpu.sync_copy(a_hbm.at[nheads + hd], adst_v.at[pl.ds(0, N)])
        adst_v[pl.ds(N, 16)] = jnp.full((16,), -30.0, F32)

        def zero(i, c):
            s_v[pl.ds(i * 16, 16)] = jnp.zeros((16,), F32)
            return c
        lax.fori_loop(0, NP // 16, zero, 0)

        base0 = q * erange

        def do_chunk(ci, c):
            b = base0 + ci * chunk
            pltpu.sync_copy(src_hbm.at[pl.ds(b, chunk)], src_v)
            pltpu.sync_copy(dst_hbm.at[pl.ds(b, chunk)], dst_v)

            def grp(g, cc):
                sv = src_v[pl.ds(g * 16, 16)]
                dv = dst_v[pl.ds(g * 16, 16)]
                e = plsc.load_gather(asrc_v, [sv]) + plsc.load_gather(adst_v, [dv])
                e = jnp.where(e > 0, e, 0.2 * e)
                w = jnp.exp(e)
                w_v[pl.ds(g * 16, 16)] = w
                plsc.addupdate_scatter(s_v, [dv], w)
                return cc
            lax.fori_loop(0, ngroups, grp, 0)
            pltpu.sync_copy(w_v, w_out.at[hd, pl.ds(b, chunk)])
            return c
        lax.fori_loop(0, nchunks, do_chunk, 0)
        pltpu.sync_copy(s_v, s_out.at[wid])

    return body


def _attn_pass(a1t, srcp, dstp, nheads, chunk):
    body = _make_attn_body(nheads, chunk)
    f = pl.kernel(
        body,
        out_type=(jax.ShapeDtypeStruct((nheads, E_PAD), F32),
                  jax.ShapeDtypeStruct((NW, NP), F32)),
        mesh=_mesh(),
        scratch_types=[
            pltpu.VMEM((NP,), F32), pltpu.VMEM((NP,), F32), pltpu.VMEM((NP,), F32),
            pltpu.VMEM((chunk,), jnp.int32), pltpu.VMEM((chunk,), jnp.int32),
            pltpu.VMEM((chunk,), F32),
        ],
        compiler_params=pltpu.CompilerParams(needs_layout_passes=False, use_tc_tiling_on_sc=False),
    )
    return f(a1t, srcp, dstp)


_UNROLL = 4


def _make_msg_body(nch, npass, nheads, chunk):
    ngroups = chunk // 16
    nchunks = E_PAD // chunk
    npairs = nchunks // 2
    assert nchunks % 2 == 0 and ngroups % _UNROLL == 0

    def body(tab_hbm, src_hbm, dst_hbm, w_hbm, out_t, *refs):
        tabs = refs[:nch]
        accs = refs[nch:2 * nch]
        bufA = refs[2 * nch:2 * nch + 3]
        bufB = refs[2 * nch + 3:2 * nch + 6]
        semA, semB = refs[2 * nch + 6:2 * nch + 8]
        wid = lax.axis_index("s") * NC + lax.axis_index("c")

        def do_pass(p, c):
            ch0 = p * (NW * nch) + wid * nch
            hd = ch0 // HID if nheads > 1 else 0

            def issue(b, buf, sem):
                pltpu.async_copy(src_hbm.at[pl.ds(b, chunk)], buf[0], sem)
                pltpu.async_copy(dst_hbm.at[pl.ds(b, chunk)], buf[1], sem)
                pltpu.async_copy(w_hbm.at[hd, pl.ds(b, chunk)], buf[2], sem)

            def drain(buf, sem):
                pltpu.make_async_copy(src_hbm.at[pl.ds(0, chunk)], buf[0], sem).wait()
                pltpu.make_async_copy(dst_hbm.at[pl.ds(0, chunk)], buf[1], sem).wait()
                pltpu.make_async_copy(w_hbm.at[hd, pl.ds(0, chunk)], buf[2], sem).wait()

            def process(buf):
                src_v, dst_v, w_v = buf

                def grp(g, ccc):
                    for u in range(_UNROLL):
                        o = (g * _UNROLL + u) * 16
                        sv = src_v[pl.ds(o, 16)]
                        dv = dst_v[pl.ds(o, 16)]
                        wv = w_v[pl.ds(o, 16)]
                        for k in range(nch):
                            rows = plsc.load_gather(tabs[k], [sv])
                            plsc.addupdate_scatter(accs[k], [dv], rows * wv)
                    return ccc
                lax.fori_loop(0, ngroups // _UNROLL, grp, 0)

            for k in range(nch):
                pltpu.sync_copy(tab_hbm.at[ch0 + k], tabs[k].at[pl.ds(0, N)])

            def zero(i, cc):
                for k in range(nch):
                    accs[k][pl.ds(i * 16, 16)] = jnp.zeros((16,), F32)
                return cc
            lax.fori_loop(0, NP // 16, zero, 0)

            issue(0, bufA, semA)

            def do_pair(ci, cc):
                b = ci * (2 * chunk)
                issue(b + chunk, bufB, semB)
                drain(bufA, semA)
                process(bufA)

                @pl.when(ci + 1 < npairs)
                def _():
                    issue(b + 2 * chunk, bufA, semA)
                drain(bufB, semB)
                process(bufB)
                return cc
            lax.fori_loop(0, npairs, do_pair, 0)

            for k in range(nch):
                pltpu.sync_copy(accs[k].at[pl.ds(0, N)], out_t.at[ch0 + k])
            return c
        lax.fori_loop(0, npass, do_pass, 0)

    return body


def _msg_pass(tab, srcp, dstp, w_e, nch, npass, nheads, chunk):
    nchan = tab.shape[0]
    body = _make_msg_body(nch, npass, nheads, chunk)
    ebuf = [pltpu.VMEM((chunk,), jnp.int32), pltpu.VMEM((chunk,), jnp.int32),
            pltpu.VMEM((chunk,), F32)]
    scr = ([pltpu.VMEM((NP,), F32)] * (2 * nch) + ebuf + ebuf +
           [pltpu.SemaphoreType.DMA, pltpu.SemaphoreType.DMA])
    f = pl.kernel(
        body,
        out_type=jax.ShapeDtypeStruct((nchan, N), F32),
        mesh=_mesh(),
        scratch_types=scr,
        compiler_params=pltpu.CompilerParams(needs_layout_passes=False, use_tc_tiling_on_sc=False),
    )
    return f(tab, srcp, dstp, w_e)


def _norm_body(acc_ref, sp_ref, bias_ref, out_ref):
    s = jnp.sum(sp_ref[...], axis=1)[:, :N]
    v = acc_ref[...] / (s + 1e-16) + bias_ref[...]
    out_ref[...] = jnp.where(v > 0, v, jnp.exp(v) - 1.0)


def _l2in_body(h1n_ref, w2_ref, ws2_ref, h2t_ref, a2t_ref):
    h1n = h1n_ref[...]
    h2t_ref[...] = lax.dot_general(w2_ref[...], h1n, (((0,), (0,)), ((), ())),
                                   preferred_element_type=F32)
    a2t_ref[...] = lax.dot_general(ws2_ref[...], h1n, (((0,), (0,)), ((), ())),
                                   preferred_element_type=F32)


def _final_body(acc2_ref, s2p_ref, bias2_ref, batch_ref, l1w_ref, l1b_ref,
                l2w_ref, l2b_ref, out_ref):
    s2 = jnp.sum(s2p_ref[...], axis=0, keepdims=True)[:, :N]
    v = acc2_ref[...] / (s2 + 1e-16) + bias2_ref[...]
    h = jnp.where(v > 0, v, jnp.exp(v) - 1.0)
    gid = lax.broadcasted_iota(jnp.int32, (N, NG), 1)
    P = (batch_ref[...] == gid).astype(F32)
    G = lax.dot_general(P, h, (((0,), (1,)), ((), ())),
                        preferred_element_type=F32)
    g1 = jnp.dot(G, l1w_ref[...], preferred_element_type=F32) + l1b_ref[...]
    g1 = jnp.where(g1 > 0, g1, jnp.exp(g1) - 1.0)
    lg = jnp.dot(g1, l2w_ref[...], preferred_element_type=F32) + l2b_ref[...]
    m = jnp.max(lg, axis=1, keepdims=True)
    lse = jnp.log(jnp.sum(jnp.exp(lg - m), axis=1, keepdims=True)) + m
    out_ref[...] = lg - lse


def kernel(x, edge_index, batch, W1, att_src1, att_dst1, bias1, W2, att_src2,
           att_dst2, bias2, lin1_W, lin1_b, lin2_W, lin2_b):
    sl = jnp.arange(N, dtype=jnp.int32)
    npad = E_PAD - E_REAL
    srcp = jnp.concatenate([edge_index[0].astype(jnp.int32), sl,
                            jnp.zeros((npad,), jnp.int32)])
    dstp = jnp.concatenate([edge_index[1].astype(jnp.int32), sl,
                            jnp.full((npad,), N, jnp.int32)])

    w1r = W1.reshape(x.shape[1], HEADS, HID)
    ws1 = jnp.concatenate([jnp.einsum('khc,hc->kh', w1r, att_src1),
                           jnp.einsum('khc,hc->kh', w1r, att_dst1)], axis=1)
    ws2 = jnp.stack([W2 @ att_src2[0], W2 @ att_dst2[0]], axis=1)

    h1t, a1t = pl.pallas_call(
        _tc_in_body,
        out_shape=(jax.ShapeDtypeStruct((HEADS * HID, N), F32),
                   jax.ShapeDtypeStruct((2 * HEADS, N), F32)),
    )(x, W1, ws1)

    w1e, s1p = _attn_pass(a1t, srcp, dstp, HEADS, 2048)
    out1t = _msg_pass(h1t, srcp, dstp, w1e, nch=4, npass=4, nheads=HEADS,
                      chunk=4096)

    h1n = pl.pallas_call(
        _norm_body,
        grid=(HEADS,),
        in_specs=[pl.BlockSpec((HID, N), lambda h: (h, 0)),
                  pl.BlockSpec((1, NW // HEADS, NP), lambda h: (h, 0, 0)),
                  pl.BlockSpec((HID, 1), lambda h: (h, 0))],
        out_specs=pl.BlockSpec((HID, N), lambda h: (h, 0)),
        out_shape=jax.ShapeDtypeStruct((HEADS * HID, N), F32),
    )(out1t, s1p.reshape(HEADS, NW // HEADS, NP), bias1.reshape(HEADS * HID, 1))

    h2t, a2t = pl.pallas_call(
        _l2in_body,
        out_shape=(jax.ShapeDtypeStruct((HID, N), F32),
                   jax.ShapeDtypeStruct((2, N), F32)),
    )(h1n, W2, ws2)

    w2e, s2p = _attn_pass(a2t, srcp, dstp, 1, 1344)
    out2t = _msg_pass(h2t, srcp, dstp, w2e, nch=2, npass=1, nheads=1,
                      chunk=4096)

    out = pl.pallas_call(
        _final_body,
        out_shape=jax.ShapeDtypeStruct((NG, 16), F32),
    )(out2t, s2p, bias2.reshape(HID, 1), batch.reshape(N, 1).astype(jnp.int32),
      lin1_W, lin1_b.reshape(1, HID), lin2_W, lin2_b.reshape(1, 16))
    return out


# speedup vs baseline: 33.4279x; 2.1323x over previous
import functools

import jax
import jax.numpy as jnp
from jax import lax
from jax.experimental import pallas as pl
from jax.experimental.pallas import tpu as pltpu
from jax.experimental.pallas import tpu_sc as plsc

N = 10000
NP = 10016
E_REAL = 330000
E_PAD = 344064
HEADS = 8
HID = 64
NG = 64
F32 = jnp.float32

NC, NS = 2, 16
NW = NC * NS


def _mesh():
    return plsc.VectorSubcoreMesh(core_axis_name="c", subcore_axis_name="s",
                                  num_cores=NC, num_subcores=NS)


def _tc_in_body(x_ref, w1_ref, ws1_ref, h1t_ref, a1t_ref):
    x = x_ref[...]
    h1t_ref[...] = lax.dot_general(w1_ref[...], x, (((0,), (1,)), ((), ())),
                                   preferred_element_type=F32)
    a1t_ref[...] = lax.dot_general(ws1_ref[...], x, (((0,), (1,)), ((), ())),
                                   preferred_element_type=F32)


def _make_attn_body(nheads, chunk):
    split = NW // nheads
    erange = E_PAD // split
    nchunks = erange // chunk
    ngroups = chunk // 16

    def body(a_hbm, src_hbm, dst_hbm, w_out, s_out, asrc_v, adst_v, s_v,
             src_v, dst_v, w_v):
        wid = lax.axis_index("s") * NC + lax.axis_index("c")
        hd = wid // split
        q = wid % split
        pltpu.sync_copy(a_hbm.at[hd], asrc_v.at[pl.ds(0, N)])
        pltpu.sync_copy(a_hbm.at[nheads + hd], adst_v.at[pl.ds(0, N)])
        adst_v[pl.ds(N, 16)] = jnp.full((16,), -30.0, F32)

        def zero(i, c):
            s_v[pl.ds(i * 16, 16)] = jnp.zeros((16,), F32)
            return c
        lax.fori_loop(0, NP // 16, zero, 0)

        base0 = q * erange

        def do_chunk(ci, c):
            b = base0 + ci * chunk
            pltpu.sync_copy(src_hbm.at[pl.ds(b, chunk)], src_v)
            pltpu.sync_copy(dst_hbm.at[pl.ds(b, chunk)], dst_v)

            def grp(g, cc):
                sv = src_v[pl.ds(g * 16, 16)]
                dv = dst_v[pl.ds(g * 16, 16)]
                e = plsc.load_gather(asrc_v, [sv]) + plsc.load_gather(adst_v, [dv])
                e = jnp.where(e > 0, e, 0.2 * e)
                w = jnp.exp(e)
                w_v[pl.ds(g * 16, 16)] = w
                plsc.addupdate_scatter(s_v, [dv], w)
                return cc
            lax.fori_loop(0, ngroups, grp, 0)
            pltpu.sync_copy(w_v, w_out.at[hd, pl.ds(b, chunk)])
            return c
        lax.fori_loop(0, nchunks, do_chunk, 0)
        pltpu.sync_copy(s_v, s_out.at[wid])

    return body


def _attn_pass(a1t, srcp, dstp, nheads, chunk):
    body = _make_attn_body(nheads, chunk)
    f = pl.kernel(
        body,
        out_type=(jax.ShapeDtypeStruct((nheads, E_PAD), F32),
                  jax.ShapeDtypeStruct((NW, NP), F32)),
        mesh=_mesh(),
        scratch_types=[
            pltpu.VMEM((NP,), F32), pltpu.VMEM((NP,), F32), pltpu.VMEM((NP,), F32),
            pltpu.VMEM((chunk,), jnp.int32), pltpu.VMEM((chunk,), jnp.int32),
            pltpu.VMEM((chunk,), F32),
        ],
        compiler_params=pltpu.CompilerParams(needs_layout_passes=False, use_tc_tiling_on_sc=False),
    )
    return f(a1t, srcp, dstp)


_UNROLL = 4


def _make_msg_body(nch, npass, nheads, chunk):
    ngroups = chunk // 16
    nchunks = E_PAD // chunk
    npairs = nchunks // 2
    assert nchunks % 2 == 0 and ngroups % _UNROLL == 0

    def body(tab_hbm, src_hbm, dst_hbm, w_hbm, out_t, *refs):
        tabs = refs[:nch]
        accs = refs[nch:2 * nch]
        bufA = refs[2 * nch:2 * nch + 3]
        bufB = refs[2 * nch + 3:2 * nch + 6]
        semA, semB = refs[2 * nch + 6:2 * nch + 8]
        wid = lax.axis_index("s") * NC + lax.axis_index("c")

        def do_pass(p, c):
            ch0 = p * (NW * nch) + wid * nch
            hd = ch0 // HID if nheads > 1 else 0

            def issue(b, buf, sem):
                pltpu.async_copy(src_hbm.at[pl.ds(b, chunk)], buf[0], sem)
                pltpu.async_copy(dst_hbm.at[pl.ds(b, chunk)], buf[1], sem)
                pltpu.async_copy(w_hbm.at[hd, pl.ds(b, chunk)], buf[2], sem)

            def drain(buf, sem):
                pltpu.make_async_copy(src_hbm.at[pl.ds(0, chunk)], buf[0], sem).wait()
                pltpu.make_async_copy(dst_hbm.at[pl.ds(0, chunk)], buf[1], sem).wait()
                pltpu.make_async_copy(w_hbm.at[hd, pl.ds(0, chunk)], buf[2], sem).wait()

            def process(buf):
                src_v, dst_v, w_v = buf

                @plsc.parallel_loop(0, ngroups, unroll=_UNROLL)
                def _(g):
                    o = g * 16
                    sv = src_v[pl.ds(o, 16)]
                    dv = dst_v[pl.ds(o, 16)]
                    wv = w_v[pl.ds(o, 16)]
                    for k in range(nch):
                        rows = plsc.load_gather(tabs[k], [sv])
                        plsc.addupdate_scatter(accs[k], [dv], rows * wv)

            for k in range(nch):
                pltpu.sync_copy(tab_hbm.at[ch0 + k], tabs[k].at[pl.ds(0, N)])

            def zero(i, cc):
                for k in range(nch):
                    accs[k][pl.ds(i * 16, 16)] = jnp.zeros((16,), F32)
                return cc
            lax.fori_loop(0, NP // 16, zero, 0)

            issue(0, bufA, semA)

            def do_pair(ci, cc):
                b = ci * (2 * chunk)
                issue(b + chunk, bufB, semB)
                drain(bufA, semA)
                process(bufA)

                @pl.when(ci + 1 < npairs)
                def _():
                    issue(b + 2 * chunk, bufA, semA)
                drain(bufB, semB)
                process(bufB)
                return cc
            lax.fori_loop(0, npairs, do_pair, 0)

            for k in range(nch):
                pltpu.sync_copy(accs[k].at[pl.ds(0, N)], out_t.at[ch0 + k])
            return c
        lax.fori_loop(0, npass, do_pass, 0)

    return body


def _msg_pass(tab, srcp, dstp, w_e, nch, npass, nheads, chunk):
    nchan = tab.shape[0]
    body = _make_msg_body(nch, npass, nheads, chunk)
    ebuf = [pltpu.VMEM((chunk,), jnp.int32), pltpu.VMEM((chunk,), jnp.int32),
            pltpu.VMEM((chunk,), F32)]
    scr = ([pltpu.VMEM((NP,), F32)] * (2 * nch) + ebuf + ebuf +
           [pltpu.SemaphoreType.DMA, pltpu.SemaphoreType.DMA])
    f = pl.kernel(
        body,
        out_type=jax.ShapeDtypeStruct((nchan, N), F32),
        mesh=_mesh(),
        scratch_types=scr,
        compiler_params=pltpu.CompilerParams(needs_layout_passes=False, use_tc_tiling_on_sc=False),
    )
    return f(tab, srcp, dstp, w_e)


def _norm_body(acc_ref, sp_ref, bias_ref, out_ref):
    s = jnp.sum(sp_ref[...], axis=1)[:, :N]
    v = acc_ref[...] / (s + 1e-16) + bias_ref[...]
    out_ref[...] = jnp.where(v > 0, v, jnp.exp(v) - 1.0)


def _l2in_body(h1n_ref, w2_ref, ws2_ref, h2t_ref, a2t_ref):
    h1n = h1n_ref[...]
    h2t_ref[...] = lax.dot_general(w2_ref[...], h1n, (((0,), (0,)), ((), ())),
                                   preferred_element_type=F32)
    a2t_ref[...] = lax.dot_general(ws2_ref[...], h1n, (((0,), (0,)), ((), ())),
                                   preferred_element_type=F32)


def _final_body(acc2_ref, s2p_ref, bias2_ref, batch_ref, l1w_ref, l1b_ref,
                l2w_ref, l2b_ref, out_ref):
    s2 = jnp.sum(s2p_ref[...], axis=0, keepdims=True)[:, :N]
    v = acc2_ref[...] / (s2 + 1e-16) + bias2_ref[...]
    h = jnp.where(v > 0, v, jnp.exp(v) - 1.0)
    gid = lax.broadcasted_iota(jnp.int32, (N, NG), 1)
    P = (batch_ref[...] == gid).astype(F32)
    G = lax.dot_general(P, h, (((0,), (1,)), ((), ())),
                        preferred_element_type=F32)
    g1 = jnp.dot(G, l1w_ref[...], preferred_element_type=F32) + l1b_ref[...]
    g1 = jnp.where(g1 > 0, g1, jnp.exp(g1) - 1.0)
    lg = jnp.dot(g1, l2w_ref[...], preferred_element_type=F32) + l2b_ref[...]
    m = jnp.max(lg, axis=1, keepdims=True)
    lse = jnp.log(jnp.sum(jnp.exp(lg - m), axis=1, keepdims=True)) + m
    out_ref[...] = lg - lse


def kernel(x, edge_index, batch, W1, att_src1, att_dst1, bias1, W2, att_src2,
           att_dst2, bias2, lin1_W, lin1_b, lin2_W, lin2_b):
    sl = jnp.arange(N, dtype=jnp.int32)
    npad = E_PAD - E_REAL
    srcp = jnp.concatenate([edge_index[0].astype(jnp.int32), sl,
                            jnp.zeros((npad,), jnp.int32)])
    dstp = jnp.concatenate([edge_index[1].astype(jnp.int32), sl,
                            jnp.full((npad,), N, jnp.int32)])

    w1r = W1.reshape(x.shape[1], HEADS, HID)
    ws1 = jnp.concatenate([jnp.einsum('khc,hc->kh', w1r, att_src1),
                           jnp.einsum('khc,hc->kh', w1r, att_dst1)], axis=1)
    ws2 = jnp.stack([W2 @ att_src2[0], W2 @ att_dst2[0]], axis=1)

    h1t, a1t = pl.pallas_call(
        _tc_in_body,
        out_shape=(jax.ShapeDtypeStruct((HEADS * HID, N), F32),
                   jax.ShapeDtypeStruct((2 * HEADS, N), F32)),
    )(x, W1, ws1)

    w1e, s1p = _attn_pass(a1t, srcp, dstp, HEADS, 2048)
    out1t = _msg_pass(h1t, srcp, dstp, w1e, nch=4, npass=4, nheads=HEADS,
                      chunk=4096)

    h1n = pl.pallas_call(
        _norm_body,
        grid=(HEADS,),
        in_specs=[pl.BlockSpec((HID, N), lambda h: (h, 0)),
                  pl.BlockSpec((1, NW // HEADS, NP), lambda h: (h, 0, 0)),
                  pl.BlockSpec((HID, 1), lambda h: (h, 0))],
        out_specs=pl.BlockSpec((HID, N), lambda h: (h, 0)),
        out_shape=jax.ShapeDtypeStruct((HEADS * HID, N), F32),
    )(out1t, s1p.reshape(HEADS, NW // HEADS, NP), bias1.reshape(HEADS * HID, 1))

    h2t, a2t = pl.pallas_call(
        _l2in_body,
        out_shape=(jax.ShapeDtypeStruct((HID, N), F32),
                   jax.ShapeDtypeStruct((2, N), F32)),
    )(h1n, W2, ws2)

    w2e, s2p = _attn_pass(a2t, srcp, dstp, 1, 1344)
    out2t = _msg_pass(h2t, srcp, dstp, w2e, nch=2, npass=1, nheads=1,
                      chunk=4096)

    out = pl.pallas_call(
        _final_body,
        out_shape=jax.ShapeDtypeStruct((NG, 16), F32),
    )(out2t, s2p, bias2.reshape(HID, 1), batch.reshape(N, 1).astype(jnp.int32),
      lin1_W, lin1_b.reshape(1, HID), lin2_W, lin2_b.reshape(1, 16))
    return out


# speedup vs baseline: 36.1310x; 1.0809x over previous
import functools

import jax
import jax.numpy as jnp
from jax import lax
from jax.experimental import pallas as pl
from jax.experimental.pallas import tpu as pltpu
from jax.experimental.pallas import tpu_sc as plsc

N = 10000
NP = 10016
E_REAL = 330000
E_PAD = 344064
HEADS = 8
HID = 64
NG = 64
F32 = jnp.float32

NC, NS = 2, 16
NW = NC * NS


def _mesh():
    return plsc.VectorSubcoreMesh(core_axis_name="c", subcore_axis_name="s",
                                  num_cores=NC, num_subcores=NS)


def _tc_in_body(x_ref, w1_ref, ws1_ref, h1t_ref, a1t_ref):
    x = x_ref[...]
    h1t_ref[...] = lax.dot_general(w1_ref[...], x, (((0,), (1,)), ((), ())),
                                   preferred_element_type=F32)
    a1t_ref[...] = lax.dot_general(ws1_ref[...], x, (((0,), (1,)), ((), ())),
                                   preferred_element_type=F32)


_UNROLL = 4


def _make_attn_body(nheads, chunk):
    split = NW // nheads
    erange = E_PAD // split
    nchunks = erange // chunk
    ngroups = chunk // 16

    def body(a_hbm, src_hbm, dst_hbm, w_out, s_out, asrc_v, adst_v, s_v,
             src_v, dst_v, w_v):
        wid = lax.axis_index("s") * NC + lax.axis_index("c")
        hd = wid // split
        q = wid % split
        pltpu.sync_copy(a_hbm.at[hd], asrc_v.at[pl.ds(0, N)])
        pltpu.sync_copy(a_hbm.at[nheads + hd], adst_v.at[pl.ds(0, N)])
        adst_v[pl.ds(N, 16)] = jnp.full((16,), -30.0, F32)

        def zero(i, c):
            s_v[pl.ds(i * 16, 16)] = jnp.zeros((16,), F32)
            return c
        lax.fori_loop(0, NP // 16, zero, 0)

        base0 = q * erange

        def do_chunk(ci, c):
            b = base0 + ci * chunk
            pltpu.sync_copy(src_hbm.at[pl.ds(b, chunk)], src_v)
            pltpu.sync_copy(dst_hbm.at[pl.ds(b, chunk)], dst_v)

            @plsc.parallel_loop(0, ngroups, unroll=_UNROLL)
            def _(g):
                sv = src_v[pl.ds(g * 16, 16)]
                dv = dst_v[pl.ds(g * 16, 16)]
                e = plsc.load_gather(asrc_v, [sv]) + plsc.load_gather(adst_v, [dv])
                e = jnp.where(e > 0, e, 0.2 * e)
                w = jnp.exp(e)
                w_v[pl.ds(g * 16, 16)] = w
                plsc.addupdate_scatter(s_v, [dv], w)
            pltpu.sync_copy(w_v, w_out.at[hd, pl.ds(b, chunk)])
            return c
        lax.fori_loop(0, nchunks, do_chunk, 0)
        pltpu.sync_copy(s_v, s_out.at[wid])

    return body


def _attn_pass(a1t, srcp, dstp, nheads, chunk):
    body = _make_attn_body(nheads, chunk)
    f = pl.kernel(
        body,
        out_type=(jax.ShapeDtypeStruct((nheads, E_PAD), F32),
                  jax.ShapeDtypeStruct((NW, NP), F32)),
        mesh=_mesh(),
        scratch_types=[
            pltpu.VMEM((NP,), F32), pltpu.VMEM((NP,), F32), pltpu.VMEM((NP,), F32),
            pltpu.VMEM((chunk,), jnp.int32), pltpu.VMEM((chunk,), jnp.int32),
            pltpu.VMEM((chunk,), F32),
        ],
        compiler_params=pltpu.CompilerParams(needs_layout_passes=False, use_tc_tiling_on_sc=False),
    )
    return f(a1t, srcp, dstp)


def _make_msg_body(nch, npass, nheads, chunk):
    ngroups = chunk // 16
    nchunks = E_PAD // chunk
    npairs = nchunks // 2
    assert nchunks % 2 == 0 and ngroups % _UNROLL == 0

    def body(tab_hbm, src_hbm, dst_hbm, w_hbm, out_t, *refs):
        tabs = refs[:nch]
        accs = refs[nch:2 * nch]
        bufA = refs[2 * nch:2 * nch + 3]
        bufB = refs[2 * nch + 3:2 * nch + 6]
        semA, semB = refs[2 * nch + 6:2 * nch + 8]
        wid = lax.axis_index("s") * NC + lax.axis_index("c")

        def do_pass(p, c):
            ch0 = p * (NW * nch) + wid * nch
            hd = ch0 // HID if nheads > 1 else 0

            def issue(b, buf, sem):
                pltpu.async_copy(src_hbm.at[pl.ds(b, chunk)], buf[0], sem)
                pltpu.async_copy(dst_hbm.at[pl.ds(b, chunk)], buf[1], sem)
                pltpu.async_copy(w_hbm.at[hd, pl.ds(b, chunk)], buf[2], sem)

            def drain(buf, sem):
                pltpu.make_async_copy(src_hbm.at[pl.ds(0, chunk)], buf[0], sem).wait()
                pltpu.make_async_copy(dst_hbm.at[pl.ds(0, chunk)], buf[1], sem).wait()
                pltpu.make_async_copy(w_hbm.at[hd, pl.ds(0, chunk)], buf[2], sem).wait()

            def process(buf):
                src_v, dst_v, w_v = buf

                @plsc.parallel_loop(0, ngroups, unroll=_UNROLL)
                def _(g):
                    o = g * 16
                    sv = src_v[pl.ds(o, 16)]
                    dv = dst_v[pl.ds(o, 16)]
                    wv = w_v[pl.ds(o, 16)]
                    for k in range(nch):
                        rows = plsc.load_gather(tabs[k], [sv])
                        plsc.addupdate_scatter(accs[k], [dv], rows * wv)

            for k in range(nch):
                pltpu.sync_copy(tab_hbm.at[ch0 + k], tabs[k].at[pl.ds(0, N)])

            def zero(i, cc):
                for k in range(nch):
                    accs[k][pl.ds(i * 16, 16)] = jnp.zeros((16,), F32)
                return cc
            lax.fori_loop(0, NP // 16, zero, 0)

            issue(0, bufA, semA)

            def do_pair(ci, cc):
                b = ci * (2 * chunk)
                issue(b + chunk, bufB, semB)
                drain(bufA, semA)
                process(bufA)

                @pl.when(ci + 1 < npairs)
                def _():
                    issue(b + 2 * chunk, bufA, semA)
                drain(bufB, semB)
                process(bufB)
                return cc
            lax.fori_loop(0, npairs, do_pair, 0)

            for k in range(nch):
                pltpu.sync_copy(accs[k].at[pl.ds(0, N)], out_t.at[ch0 + k])
            return c
        lax.fori_loop(0, npass, do_pass, 0)

    return body


def _msg_pass(tab, srcp, dstp, w_e, nch, npass, nheads, chunk):
    nchan = tab.shape[0]
    body = _make_msg_body(nch, npass, nheads, chunk)
    ebuf = [pltpu.VMEM((chunk,), jnp.int32), pltpu.VMEM((chunk,), jnp.int32),
            pltpu.VMEM((chunk,), F32)]
    scr = ([pltpu.VMEM((NP,), F32)] * (2 * nch) + ebuf + ebuf +
           [pltpu.SemaphoreType.DMA, pltpu.SemaphoreType.DMA])
    f = pl.kernel(
        body,
        out_type=jax.ShapeDtypeStruct((nchan, N), F32),
        mesh=_mesh(),
        scratch_types=scr,
        compiler_params=pltpu.CompilerParams(needs_layout_passes=False, use_tc_tiling_on_sc=False),
    )
    return f(tab, srcp, dstp, w_e)


def _norm_body(acc_ref, sp_ref, bias_ref, out_ref):
    s = jnp.sum(sp_ref[...], axis=1)[:, :N]
    v = acc_ref[...] / (s + 1e-16) + bias_ref[...]
    out_ref[...] = jnp.where(v > 0, v, jnp.exp(v) - 1.0)


def _l2in_body(h1n_ref, w2_ref, ws2_ref, h2t_ref, a2t_ref):
    h1n = h1n_ref[...]
    h2t_ref[...] = lax.dot_general(w2_ref[...], h1n, (((0,), (0,)), ((), ())),
                                   preferred_element_type=F32)
    a2t_ref[...] = lax.dot_general(ws2_ref[...], h1n, (((0,), (0,)), ((), ())),
                                   preferred_element_type=F32)


def _final_body(acc2_ref, s2p_ref, bias2_ref, batch_ref, l1w_ref, l1b_ref,
                l2w_ref, l2b_ref, out_ref):
    s2 = jnp.sum(s2p_ref[...], axis=0, keepdims=True)[:, :N]
    v = acc2_ref[...] / (s2 + 1e-16) + bias2_ref[...]
    h = jnp.where(v > 0, v, jnp.exp(v) - 1.0)
    gid = lax.broadcasted_iota(jnp.int32, (N, NG), 1)
    P = (batch_ref[...] == gid).astype(F32)
    G = lax.dot_general(P, h, (((0,), (1,)), ((), ())),
                        preferred_element_type=F32)
    g1 = jnp.dot(G, l1w_ref[...], preferred_element_type=F32) + l1b_ref[...]
    g1 = jnp.where(g1 > 0, g1, jnp.exp(g1) - 1.0)
    lg = jnp.dot(g1, l2w_ref[...], preferred_element_type=F32) + l2b_ref[...]
    m = jnp.max(lg, axis=1, keepdims=True)
    lse = jnp.log(jnp.sum(jnp.exp(lg - m), axis=1, keepdims=True)) + m
    out_ref[...] = lg - lse


def kernel(x, edge_index, batch, W1, att_src1, att_dst1, bias1, W2, att_src2,
           att_dst2, bias2, lin1_W, lin1_b, lin2_W, lin2_b):
    sl = jnp.arange(N, dtype=jnp.int32)
    npad = E_PAD - E_REAL
    srcp = jnp.concatenate([edge_index[0].astype(jnp.int32), sl,
                            jnp.zeros((npad,), jnp.int32)])
    dstp = jnp.concatenate([edge_index[1].astype(jnp.int32), sl,
                            jnp.full((npad,), N, jnp.int32)])

    w1r = W1.reshape(x.shape[1], HEADS, HID)
    ws1 = jnp.concatenate([jnp.einsum('khc,hc->kh', w1r, att_src1),
                           jnp.einsum('khc,hc->kh', w1r, att_dst1)], axis=1)
    ws2 = jnp.stack([W2 @ att_src2[0], W2 @ att_dst2[0]], axis=1)

    h1t, a1t = pl.pallas_call(
        _tc_in_body,
        out_shape=(jax.ShapeDtypeStruct((HEADS * HID, N), F32),
                   jax.ShapeDtypeStruct((2 * HEADS, N), F32)),
    )(x, W1, ws1)

    w1e, s1p = _attn_pass(a1t, srcp, dstp, HEADS, 4096)
    out1t = _msg_pass(h1t, srcp, dstp, w1e, nch=4, npass=4, nheads=HEADS,
                      chunk=4096)

    h1n = pl.pallas_call(
        _norm_body,
        grid=(HEADS,),
        in_specs=[pl.BlockSpec((HID, N), lambda h: (h, 0)),
                  pl.BlockSpec((1, NW // HEADS, NP), lambda h: (h, 0, 0)),
                  pl.BlockSpec((HID, 1), lambda h: (h, 0))],
        out_specs=pl.BlockSpec((HID, N), lambda h: (h, 0)),
        out_shape=jax.ShapeDtypeStruct((HEADS * HID, N), F32),
    )(out1t, s1p.reshape(HEADS, NW // HEADS, NP), bias1.reshape(HEADS * HID, 1))

    h2t, a2t = pl.pallas_call(
        _l2in_body,
        out_shape=(jax.ShapeDtypeStruct((HID, N), F32),
                   jax.ShapeDtypeStruct((2, N), F32)),
    )(h1n, W2, ws2)

    w2e, s2p = _attn_pass(a2t, srcp, dstp, 1, 2688)
    out2t = _msg_pass(h2t, srcp, dstp, w2e, nch=2, npass=1, nheads=1,
                      chunk=4096)

    out = pl.pallas_call(
        _final_body,
        out_shape=jax.ShapeDtypeStruct((NG, 16), F32),
    )(out2t, s2p, bias2.reshape(HID, 1), batch.reshape(N, 1).astype(jnp.int32),
      lin1_W, lin1_b.reshape(1, HID), lin2_W, lin2_b.reshape(1, 16))
    return out


# speedup vs baseline: 38.0896x; 1.0542x over previous
import functools

import jax
import jax.numpy as jnp
from jax import lax
from jax.experimental import pallas as pl
from jax.experimental.pallas import tpu as pltpu
from jax.experimental.pallas import tpu_sc as plsc

N = 10000
NP = 10016
E_REAL = 330000
E_PAD = 344064
HEADS = 8
HID = 64
NG = 64
F32 = jnp.float32

NC, NS = 2, 16
NW = NC * NS


def _mesh():
    return plsc.VectorSubcoreMesh(core_axis_name="c", subcore_axis_name="s",
                                  num_cores=NC, num_subcores=NS)


def _tc_in_body(x_ref, w1_ref, ws1_ref, h1t_ref, a1t_ref):
    x = x_ref[...]
    h1t_ref[...] = lax.dot_general(w1_ref[...], x, (((0,), (1,)), ((), ())),
                                   preferred_element_type=F32)
    a1t_ref[...] = lax.dot_general(ws1_ref[...], x, (((0,), (1,)), ((), ())),
                                   preferred_element_type=F32)


_UNROLL = 8


def _make_attn_body(nheads, chunk):
    split = NW // nheads
    erange = E_PAD // split
    nchunks = erange // chunk
    ngroups = chunk // 16

    def body(a_hbm, pk_hbm, w_out, s_out, asrc_v, adst_v, s_v, pk_v, w_v):
        wid = lax.axis_index("s") * NC + lax.axis_index("c")
        hd = wid // split
        q = wid % split
        pltpu.sync_copy(a_hbm.at[hd], asrc_v.at[pl.ds(0, N)])
        pltpu.sync_copy(a_hbm.at[nheads + hd], adst_v.at[pl.ds(0, N)])
        adst_v[pl.ds(N, 16)] = jnp.full((16,), -30.0, F32)

        def zero(i, c):
            s_v[pl.ds(i * 16, 16)] = jnp.zeros((16,), F32)
            return c
        lax.fori_loop(0, NP // 16, zero, 0)

        base0 = q * erange

        def do_chunk(ci, c):
            b = base0 + ci * chunk
            pltpu.sync_copy(pk_hbm.at[pl.ds(b, chunk)], pk_v)

            @plsc.parallel_loop(0, ngroups, unroll=_UNROLL)
            def _(g):
                pkv = pk_v[pl.ds(g * 16, 16)]
                sv = pkv & 0xFFFF
                dv = lax.shift_right_logical(pkv, 16)
                e = plsc.load_gather(asrc_v, [sv]) + plsc.load_gather(adst_v, [dv])
                e = jnp.where(e > 0, e, 0.2 * e)
                w = jnp.exp(e)
                w_v[pl.ds(g * 16, 16)] = w
                plsc.addupdate_scatter(s_v, [dv], w)
            pltpu.sync_copy(w_v, w_out.at[hd, pl.ds(b, chunk)])
            return c
        lax.fori_loop(0, nchunks, do_chunk, 0)
        pltpu.sync_copy(s_v, s_out.at[wid])

    return body


def _attn_pass(a1t, pk, nheads, chunk):
    body = _make_attn_body(nheads, chunk)
    f = pl.kernel(
        body,
        out_type=(jax.ShapeDtypeStruct((nheads, E_PAD), F32),
                  jax.ShapeDtypeStruct((NW, NP), F32)),
        mesh=_mesh(),
        scratch_types=[
            pltpu.VMEM((NP,), F32), pltpu.VMEM((NP,), F32), pltpu.VMEM((NP,), F32),
            pltpu.VMEM((chunk,), jnp.int32), pltpu.VMEM((chunk,), F32),
        ],
        compiler_params=pltpu.CompilerParams(needs_layout_passes=False, use_tc_tiling_on_sc=False),
    )
    return f(a1t, pk)


def _make_msg_body(nch, npass, nheads, chunk):
    ngroups = chunk // 16
    nchunks = E_PAD // chunk
    npairs = nchunks // 2
    assert nchunks % 2 == 0 and ngroups % _UNROLL == 0

    def body(tab_hbm, pk_hbm, w_hbm, out_t, *refs):
        tabs = refs[:nch]
        accs = refs[nch:2 * nch]
        bufA = refs[2 * nch:2 * nch + 2]
        bufB = refs[2 * nch + 2:2 * nch + 4]
        semA, semB = refs[2 * nch + 4:2 * nch + 6]
        wid = lax.axis_index("s") * NC + lax.axis_index("c")

        def do_pass(p, c):
            ch0 = p * (NW * nch) + wid * nch
            hd = ch0 // HID if nheads > 1 else 0

            def issue(b, buf, sem):
                pltpu.async_copy(pk_hbm.at[pl.ds(b, chunk)], buf[0], sem)
                pltpu.async_copy(w_hbm.at[hd, pl.ds(b, chunk)], buf[1], sem)

            def drain(buf, sem):
                pltpu.make_async_copy(pk_hbm.at[pl.ds(0, chunk)], buf[0], sem).wait()
                pltpu.make_async_copy(w_hbm.at[hd, pl.ds(0, chunk)], buf[1], sem).wait()

            def process(buf):
                pk_v, w_v = buf

                @plsc.parallel_loop(0, ngroups, unroll=_UNROLL)
                def _(g):
                    o = g * 16
                    pkv = pk_v[pl.ds(o, 16)]
                    sv = pkv & 0xFFFF
                    dv = lax.shift_right_logical(pkv, 16)
                    wv = w_v[pl.ds(o, 16)]
                    for k in range(nch):
                        rows = plsc.load_gather(tabs[k], [sv])
                        plsc.addupdate_scatter(accs[k], [dv], rows * wv)

            for k in range(nch):
                pltpu.sync_copy(tab_hbm.at[ch0 + k], tabs[k].at[pl.ds(0, N)])

            def zero(i, cc):
                for k in range(nch):
                    accs[k][pl.ds(i * 16, 16)] = jnp.zeros((16,), F32)
                return cc
            lax.fori_loop(0, NP // 16, zero, 0)

            issue(0, bufA, semA)

            def do_pair(ci, cc):
                b = ci * (2 * chunk)
                issue(b + chunk, bufB, semB)
                drain(bufA, semA)
                process(bufA)

                @pl.when(ci + 1 < npairs)
                def _():
                    issue(b + 2 * chunk, bufA, semA)
                drain(bufB, semB)
                process(bufB)
                return cc
            lax.fori_loop(0, npairs, do_pair, 0)

            for k in range(nch):
                pltpu.sync_copy(accs[k].at[pl.ds(0, N)], out_t.at[ch0 + k])
            return c
        lax.fori_loop(0, npass, do_pass, 0)

    return body


def _msg_pass(tab, pk, w_e, nch, npass, nheads, chunk):
    nchan = tab.shape[0]
    body = _make_msg_body(nch, npass, nheads, chunk)
    ebuf = [pltpu.VMEM((chunk,), jnp.int32), pltpu.VMEM((chunk,), F32)]
    scr = ([pltpu.VMEM((NP,), F32)] * (2 * nch) + ebuf + ebuf +
           [pltpu.SemaphoreType.DMA, pltpu.SemaphoreType.DMA])
    f = pl.kernel(
        body,
        out_type=jax.ShapeDtypeStruct((nchan, N), F32),
        mesh=_mesh(),
        scratch_types=scr,
        compiler_params=pltpu.CompilerParams(needs_layout_passes=False, use_tc_tiling_on_sc=False),
    )
    return f(tab, pk, w_e)


def _norm_body(acc_ref, sp_ref, bias_ref, out_ref):
    s = jnp.sum(sp_ref[...], axis=1)[:, :N]
    v = acc_ref[...] / (s + 1e-16) + bias_ref[...]
    out_ref[...] = jnp.where(v > 0, v, jnp.exp(v) - 1.0)


def _l2in_body(h1n_ref, w2_ref, ws2_ref, h2t_ref, a2t_ref):
    h1n = h1n_ref[...]
    h2t_ref[...] = lax.dot_general(w2_ref[...], h1n, (((0,), (0,)), ((), ())),
                                   preferred_element_type=F32)
    a2t_ref[...] = lax.dot_general(ws2_ref[...], h1n, (((0,), (0,)), ((), ())),
                                   preferred_element_type=F32)


def _final_body(acc2_ref, s2p_ref, bias2_ref, batch_ref, l1w_ref, l1b_ref,
                l2w_ref, l2b_ref, out_ref):
    s2 = jnp.sum(s2p_ref[...], axis=0, keepdims=True)[:, :N]
    v = acc2_ref[...] / (s2 + 1e-16) + bias2_ref[...]
    h = jnp.where(v > 0, v, jnp.exp(v) - 1.0)
    gid = lax.broadcasted_iota(jnp.int32, (N, NG), 1)
    P = (batch_ref[...] == gid).astype(F32)
    G = lax.dot_general(P, h, (((0,), (1,)), ((), ())),
                        preferred_element_type=F32)
    g1 = jnp.dot(G, l1w_ref[...], preferred_element_type=F32) + l1b_ref[...]
    g1 = jnp.where(g1 > 0, g1, jnp.exp(g1) - 1.0)
    lg = jnp.dot(g1, l2w_ref[...], preferred_element_type=F32) + l2b_ref[...]
    m = jnp.max(lg, axis=1, keepdims=True)
    lse = jnp.log(jnp.sum(jnp.exp(lg - m), axis=1, keepdims=True)) + m
    out_ref[...] = lg - lse


def kernel(x, edge_index, batch, W1, att_src1, att_dst1, bias1, W2, att_src2,
           att_dst2, bias2, lin1_W, lin1_b, lin2_W, lin2_b):
    sl = jnp.arange(N, dtype=jnp.int32)
    npad = E_PAD - E_REAL
    srcp = jnp.concatenate([edge_index[0].astype(jnp.int32), sl,
                            jnp.zeros((npad,), jnp.int32)])
    dstp = jnp.concatenate([edge_index[1].astype(jnp.int32), sl,
                            jnp.full((npad,), N, jnp.int32)])
    pk = srcp | (dstp << 16)

    w1r = W1.reshape(x.shape[1], HEADS, HID)
    ws1 = jnp.concatenate([jnp.einsum('khc,hc->kh', w1r, att_src1),
                           jnp.einsum('khc,hc->kh', w1r, att_dst1)], axis=1)
    ws2 = jnp.stack([W2 @ att_src2[0], W2 @ att_dst2[0]], axis=1)

    h1t, a1t = pl.pallas_call(
        _tc_in_body,
        out_shape=(jax.ShapeDtypeStruct((HEADS * HID, N), F32),
                   jax.ShapeDtypeStruct((2 * HEADS, N), F32)),
    )(x, W1, ws1)

    w1e, s1p = _attn_pass(a1t, pk, HEADS, 4096)
    out1t = _msg_pass(h1t, pk, w1e, nch=4, npass=4, nheads=HEADS,
                      chunk=4096)

    h1n = pl.pallas_call(
        _norm_body,
        grid=(HEADS,),
        in_specs=[pl.BlockSpec((HID, N), lambda h: (h, 0)),
                  pl.BlockSpec((1, NW // HEADS, NP), lambda h: (h, 0, 0)),
                  pl.BlockSpec((HID, 1), lambda h: (h, 0))],
        out_specs=pl.BlockSpec((HID, N), lambda h: (h, 0)),
        out_shape=jax.ShapeDtypeStruct((HEADS * HID, N), F32),
    )(out1t, s1p.reshape(HEADS, NW // HEADS, NP), bias1.reshape(HEADS * HID, 1))

    h2t, a2t = pl.pallas_call(
        _l2in_body,
        out_shape=(jax.ShapeDtypeStruct((HID, N), F32),
                   jax.ShapeDtypeStruct((2, N), F32)),
    )(h1n, W2, ws2)

    w2e, s2p = _attn_pass(a2t, pk, 1, 2688)
    out2t = _msg_pass(h2t, pk, w2e, nch=2, npass=1, nheads=1,
                      chunk=4096)

    out = pl.pallas_call(
        _final_body,
        out_shape=jax.ShapeDtypeStruct((NG, 16), F32),
    )(out2t, s2p, bias2.reshape(HID, 1), batch.reshape(N, 1).astype(jnp.int32),
      lin1_W, lin1_b.reshape(1, HID), lin2_W, lin2_b.reshape(1, 16))
    return out


# speedup vs baseline: 41.4340x; 1.0878x over previous
import functools

import jax
import jax.numpy as jnp
from jax import lax
from jax.experimental import pallas as pl
from jax.experimental.pallas import tpu as pltpu
from jax.experimental.pallas import tpu_sc as plsc

N = 10000
NP = 10016
E_REAL = 330000
E_PAD = 344064
HEADS = 8
HID = 64
NG = 64
F32 = jnp.float32

NC, NS = 2, 16
NW = NC * NS


def _mesh():
    return plsc.VectorSubcoreMesh(core_axis_name="c", subcore_axis_name="s",
                                  num_cores=NC, num_subcores=NS)


def _tc_in_body(x_ref, w1_ref, ws1_ref, h1t_ref, a1t_ref):
    x = x_ref[...]
    h1t_ref[...] = lax.dot_general(w1_ref[...], x, (((0,), (1,)), ((), ())),
                                   preferred_element_type=F32)
    a1t_ref[...] = lax.dot_general(ws1_ref[...], x, (((0,), (1,)), ((), ())),
                                   preferred_element_type=F32)


_UNROLL = 8


def _make_attn_body(nheads, chunk):
    split = NW // nheads
    erange = E_PAD // split
    nchunks = erange // chunk
    ngroups = chunk // 16

    def body(a_hbm, pk_hbm, w_out, s_out, asrc_v, adst_v, s_v, pk_v, w_v):
        wid = lax.axis_index("s") * NC + lax.axis_index("c")
        hd = wid // split
        q = wid % split
        pltpu.sync_copy(a_hbm.at[hd], asrc_v.at[pl.ds(0, N)])
        pltpu.sync_copy(a_hbm.at[nheads + hd], adst_v.at[pl.ds(0, N)])
        adst_v[pl.ds(N, 16)] = jnp.full((16,), -30.0, F32)

        def zero(i, c):
            s_v[pl.ds(i * 16, 16)] = jnp.zeros((16,), F32)
            return c
        lax.fori_loop(0, NP // 16, zero, 0)

        base0 = q * erange

        def do_chunk(ci, c):
            b = base0 + ci * chunk
            pltpu.sync_copy(pk_hbm.at[pl.ds(b, chunk)], pk_v)

            @plsc.parallel_loop(0, ngroups, unroll=_UNROLL)
            def _(g):
                pkv = pk_v[pl.ds(g * 16, 16)]
                sv = pkv & 0xFFFF
                dv = lax.shift_right_logical(pkv, 16)
                e = plsc.load_gather(asrc_v, [sv]) + plsc.load_gather(adst_v, [dv])
                e = jnp.where(e > 0, e, 0.2 * e)
                w = jnp.exp(e)
                w_v[pl.ds(g * 16, 16)] = w
                plsc.addupdate_scatter(s_v, [dv], w)
            pltpu.sync_copy(w_v, w_out.at[hd, pl.ds(b, chunk)])
            return c
        lax.fori_loop(0, nchunks, do_chunk, 0)
        pltpu.sync_copy(s_v, s_out.at[wid])

    return body


def _attn_pass(a1t, pk, nheads, chunk):
    body = _make_attn_body(nheads, chunk)
    f = pl.kernel(
        body,
        out_type=(jax.ShapeDtypeStruct((nheads, E_PAD), F32),
                  jax.ShapeDtypeStruct((NW, NP), F32)),
        mesh=_mesh(),
        scratch_types=[
            pltpu.VMEM((NP,), F32), pltpu.VMEM((NP,), F32), pltpu.VMEM((NP,), F32),
            pltpu.VMEM((chunk,), jnp.int32), pltpu.VMEM((chunk,), F32),
        ],
        compiler_params=pltpu.CompilerParams(needs_layout_passes=False, use_tc_tiling_on_sc=False),
    )
    return f(a1t, pk)


def _pack_pairs(t):
    u = lax.bitcast_convert_type(t.astype(jnp.bfloat16), jnp.uint16).astype(jnp.uint32)
    return (u[0::2, :] | (u[1::2, :] << 16)).astype(jnp.int32)


def _make_msg_body(npair, npass, nheads, chunk):
    nch = 2 * npair
    ngroups = chunk // 16
    nchunks = E_PAD // chunk
    npairs = nchunks // 2
    assert nchunks % 2 == 0 and ngroups % _UNROLL == 0

    def body(tab_hbm, pk_hbm, w_hbm, out_t, *refs):
        tabs = refs[:npair]
        accs = refs[npair:npair + nch]
        bufA = refs[npair + nch:npair + nch + 2]
        bufB = refs[npair + nch + 2:npair + nch + 4]
        semA, semB = refs[npair + nch + 4:npair + nch + 6]
        wid = lax.axis_index("s") * NC + lax.axis_index("c")

        def do_pass(p, c):
            ch0 = p * (NW * nch) + wid * nch
            hd = ch0 // HID if nheads > 1 else 0

            def issue(b, buf, sem):
                pltpu.async_copy(pk_hbm.at[pl.ds(b, chunk)], buf[0], sem)
                pltpu.async_copy(w_hbm.at[hd, pl.ds(b, chunk)], buf[1], sem)

            def drain(buf, sem):
                pltpu.make_async_copy(pk_hbm.at[pl.ds(0, chunk)], buf[0], sem).wait()
                pltpu.make_async_copy(w_hbm.at[hd, pl.ds(0, chunk)], buf[1], sem).wait()

            def process(buf):
                pk_v, w_v = buf

                @plsc.parallel_loop(0, ngroups, unroll=_UNROLL)
                def _(g):
                    o = g * 16
                    pkv = pk_v[pl.ds(o, 16)]
                    sv = pkv & 0xFFFF
                    dv = lax.shift_right_logical(pkv, 16)
                    wv = w_v[pl.ds(o, 16)]
                    for j in range(npair):
                        pr = plsc.load_gather(tabs[j], [sv])
                        va, vb = plsc.unpack(plsc.bitcast(pr, jnp.bfloat16),
                                             format=plsc.PackFormat.INTERLEAVED,
                                             preferred_element_type=F32)
                        plsc.addupdate_scatter(accs[2 * j], [dv], va * wv)
                        plsc.addupdate_scatter(accs[2 * j + 1], [dv], vb * wv)

            for j in range(npair):
                pltpu.sync_copy(tab_hbm.at[ch0 // 2 + j], tabs[j].at[pl.ds(0, N)])

            def zero(i, cc):
                for k in range(nch):
                    accs[k][pl.ds(i * 16, 16)] = jnp.zeros((16,), F32)
                return cc
            lax.fori_loop(0, NP // 16, zero, 0)

            issue(0, bufA, semA)

            def do_pair(ci, cc):
                b = ci * (2 * chunk)
                issue(b + chunk, bufB, semB)
                drain(bufA, semA)
                process(bufA)

                @pl.when(ci + 1 < npairs)
                def _():
                    issue(b + 2 * chunk, bufA, semA)
                drain(bufB, semB)
                process(bufB)
                return cc
            lax.fori_loop(0, npairs, do_pair, 0)

            for k in range(nch):
                pltpu.sync_copy(accs[k].at[pl.ds(0, N)], out_t.at[ch0 + k])
            return c
        lax.fori_loop(0, npass, do_pass, 0)

    return body


def _msg_pass(tab, pk, w_e, npair, npass, nheads, chunk):
    nchan = 2 * tab.shape[0]
    body = _make_msg_body(npair, npass, nheads, chunk)
    ebuf = [pltpu.VMEM((chunk,), jnp.int32), pltpu.VMEM((chunk,), F32)]
    scr = ([pltpu.VMEM((NP,), jnp.int32)] * npair +
           [pltpu.VMEM((NP,), F32)] * (2 * npair) + ebuf + ebuf +
           [pltpu.SemaphoreType.DMA, pltpu.SemaphoreType.DMA])
    f = pl.kernel(
        body,
        out_type=jax.ShapeDtypeStruct((nchan, N), F32),
        mesh=_mesh(),
        scratch_types=scr,
        compiler_params=pltpu.CompilerParams(needs_layout_passes=False, use_tc_tiling_on_sc=False),
    )
    return f(tab, pk, w_e)


def _norm_body(acc_ref, sp_ref, bias_ref, out_ref):
    s = jnp.sum(sp_ref[...], axis=1)[:, :N]
    v = acc_ref[...] / (s + 1e-16) + bias_ref[...]
    out_ref[...] = jnp.where(v > 0, v, jnp.exp(v) - 1.0)


def _l2in_body(h1n_ref, w2_ref, ws2_ref, h2t_ref, a2t_ref):
    h1n = h1n_ref[...]
    h2t_ref[...] = lax.dot_general(w2_ref[...], h1n, (((0,), (0,)), ((), ())),
                                   preferred_element_type=F32)
    a2t_ref[...] = lax.dot_general(ws2_ref[...], h1n, (((0,), (0,)), ((), ())),
                                   preferred_element_type=F32)


def _final_body(acc2_ref, s2p_ref, bias2_ref, batch_ref, l1w_ref, l1b_ref,
                l2w_ref, l2b_ref, out_ref):
    s2 = jnp.sum(s2p_ref[...], axis=0, keepdims=True)[:, :N]
    v = acc2_ref[...] / (s2 + 1e-16) + bias2_ref[...]
    h = jnp.where(v > 0, v, jnp.exp(v) - 1.0)
    gid = lax.broadcasted_iota(jnp.int32, (N, NG), 1)
    P = (batch_ref[...] == gid).astype(F32)
    G = lax.dot_general(P, h, (((0,), (1,)), ((), ())),
                        preferred_element_type=F32)
    g1 = jnp.dot(G, l1w_ref[...], preferred_element_type=F32) + l1b_ref[...]
    g1 = jnp.where(g1 > 0, g1, jnp.exp(g1) - 1.0)
    lg = jnp.dot(g1, l2w_ref[...], preferred_element_type=F32) + l2b_ref[...]
    m = jnp.max(lg, axis=1, keepdims=True)
    lse = jnp.log(jnp.sum(jnp.exp(lg - m), axis=1, keepdims=True)) + m
    out_ref[...] = lg - lse


def kernel(x, edge_index, batch, W1, att_src1, att_dst1, bias1, W2, att_src2,
           att_dst2, bias2, lin1_W, lin1_b, lin2_W, lin2_b):
    sl = jnp.arange(N, dtype=jnp.int32)
    npad = E_PAD - E_REAL
    srcp = jnp.concatenate([edge_index[0].astype(jnp.int32), sl,
                            jnp.zeros((npad,), jnp.int32)])
    dstp = jnp.concatenate([edge_index[1].astype(jnp.int32), sl,
                            jnp.full((npad,), N, jnp.int32)])
    pk = srcp | (dstp << 16)

    w1r = W1.reshape(x.shape[1], HEADS, HID)
    ws1 = jnp.concatenate([jnp.einsum('khc,hc->kh', w1r, att_src1),
                           jnp.einsum('khc,hc->kh', w1r, att_dst1)], axis=1)
    ws2 = jnp.stack([W2 @ att_src2[0], W2 @ att_dst2[0]], axis=1)

    h1t, a1t = pl.pallas_call(
        _tc_in_body,
        out_shape=(jax.ShapeDtypeStruct((HEADS * HID, N), F32),
                   jax.ShapeDtypeStruct((2 * HEADS, N), F32)),
    )(x, W1, ws1)

    w1e, s1p = _attn_pass(a1t, pk, HEADS, 4096)
    out1t = _msg_pass(_pack_pairs(h1t), pk, w1e, npair=4, npass=2,
                      nheads=HEADS, chunk=2048)

    h1n = pl.pallas_call(
        _norm_body,
        grid=(HEADS,),
        in_specs=[pl.BlockSpec((HID, N), lambda h: (h, 0)),
                  pl.BlockSpec((1, NW // HEADS, NP), lambda h: (h, 0, 0)),
                  pl.BlockSpec((HID, 1), lambda h: (h, 0))],
        out_specs=pl.BlockSpec((HID, N), lambda h: (h, 0)),
        out_shape=jax.ShapeDtypeStruct((HEADS * HID, N), F32),
    )(out1t, s1p.reshape(HEADS, NW // HEADS, NP), bias1.reshape(HEADS * HID, 1))

    h2t, a2t = pl.pallas_call(
        _l2in_body,
        out_shape=(jax.ShapeDtypeStruct((HID, N), F32),
                   jax.ShapeDtypeStruct((2, N), F32)),
    )(h1n, W2, ws2)

    w2e, s2p = _attn_pass(a2t, pk, 1, 2688)
    out2t = _msg_pass(_pack_pairs(h2t), pk, w2e, npair=1, npass=1, nheads=1,
                      chunk=4096)

    out = pl.pallas_call(
        _final_body,
        out_shape=jax.ShapeDtypeStruct((NG, 16), F32),
    )(out2t, s2p, bias2.reshape(HID, 1), batch.reshape(N, 1).astype(jnp.int32),
      lin1_W, lin1_b.reshape(1, HID), lin2_W, lin2_b.reshape(1, 16))
    return out


# speedup vs baseline: 41.7906x; 1.0086x over previous
import functools

import jax
import jax.numpy as jnp
from jax import lax
from jax.experimental import pallas as pl
from jax.experimental.pallas import tpu as pltpu
from jax.experimental.pallas import tpu_sc as plsc

N = 10000
NP = 10016
E_REAL = 330000
E_PAD = 344064
HEADS = 8
HID = 64
NG = 64
F32 = jnp.float32

NC, NS = 2, 16
NW = NC * NS


def _mesh():
    return plsc.VectorSubcoreMesh(core_axis_name="c", subcore_axis_name="s",
                                  num_cores=NC, num_subcores=NS)


def _tc_in_body(x_ref, w1_ref, ws1_ref, h1t_ref, a1t_ref):
    x = x_ref[...]
    h1t_ref[...] = lax.dot_general(w1_ref[...], x, (((0,), (1,)), ((), ())),
                                   preferred_element_type=F32)
    a1t_ref[...] = lax.dot_general(ws1_ref[...], x, (((0,), (1,)), ((), ())),
                                   preferred_element_type=F32)


_UNROLL = 8


def _make_attn_body(nheads, chunk):
    split = NW // nheads
    erange = E_PAD // split
    nchunks = erange // chunk
    ngroups = chunk // 16

    def body(a_hbm, pk_hbm, w_out, s_out, asrc_v, adst_v, s_v, pk_v, w_v):
        wid = lax.axis_index("s") * NC + lax.axis_index("c")
        hd = wid // split
        q = wid % split
        pltpu.sync_copy(a_hbm.at[hd], asrc_v.at[pl.ds(0, N)])
        pltpu.sync_copy(a_hbm.at[nheads + hd], adst_v.at[pl.ds(0, N)])
        adst_v[pl.ds(N, 16)] = jnp.full((16,), -30.0, F32)

        def zero(i, c):
            s_v[pl.ds(i * 16, 16)] = jnp.zeros((16,), F32)
            return c
        lax.fori_loop(0, NP // 16, zero, 0)

        base0 = q * erange

        def do_chunk(ci, c):
            b = base0 + ci * chunk
            pltpu.sync_copy(pk_hbm.at[pl.ds(b, chunk)], pk_v)

            @plsc.parallel_loop(0, ngroups, unroll=_UNROLL)
            def _(g):
                pkv = pk_v[pl.ds(g * 16, 16)]
                sv = pkv & 0xFFFF
                dv = lax.shift_right_logical(pkv, 16)
                e = plsc.load_gather(asrc_v, [sv]) + plsc.load_gather(adst_v, [dv])
                e = jnp.where(e > 0, e, 0.2 * e)
                w = jnp.exp(e)
                w_v[pl.ds(g * 16, 16)] = w
                plsc.addupdate_scatter(s_v, [dv], w)
            pltpu.sync_copy(w_v, w_out.at[hd, pl.ds(b, chunk)])
            return c
        lax.fori_loop(0, nchunks, do_chunk, 0)
        pltpu.sync_copy(s_v, s_out.at[wid])

    return body


def _attn_pass(a1t, pk, nheads, chunk):
    body = _make_attn_body(nheads, chunk)
    f = pl.kernel(
        body,
        out_type=(jax.ShapeDtypeStruct((nheads, E_PAD), F32),
                  jax.ShapeDtypeStruct((NW, NP), F32)),
        mesh=_mesh(),
        scratch_types=[
            pltpu.VMEM((NP,), F32), pltpu.VMEM((NP,), F32), pltpu.VMEM((NP,), F32),
            pltpu.VMEM((chunk,), jnp.int32), pltpu.VMEM((chunk,), F32),
        ],
        compiler_params=pltpu.CompilerParams(needs_layout_passes=False, use_tc_tiling_on_sc=False),
    )
    return f(a1t, pk)


def _pack_pairs(t):
    u = lax.bitcast_convert_type(t.astype(jnp.bfloat16), jnp.uint16).astype(jnp.uint32)
    return (u[0::2, :] | (u[1::2, :] << 16)).astype(jnp.int32)


def _make_msg_body(npair, npass, nheads, chunk):
    nch = 2 * npair
    ngroups = chunk // 16
    nchunks = E_PAD // chunk
    npairs = nchunks // 2
    assert nchunks % 2 == 0 and ngroups % _UNROLL == 0

    def body(tab_hbm, pk_hbm, w_hbm, out_t, *refs):
        tabs = refs[:npair]
        accs = refs[npair:npair + nch]
        bufA = refs[npair + nch:npair + nch + 2]
        bufB = refs[npair + nch + 2:npair + nch + 4]
        semA, semB = refs[npair + nch + 4:npair + nch + 6]
        wid = lax.axis_index("s") * NC + lax.axis_index("c")

        def do_pass(p, c):
            ch0 = p * (NW * nch) + wid * nch
            hd = ch0 // HID if nheads > 1 else 0

            def issue(b, buf, sem):
                pltpu.async_copy(pk_hbm.at[pl.ds(b, chunk)], buf[0], sem)
                pltpu.async_copy(w_hbm.at[hd, pl.ds(b, chunk)], buf[1], sem)

            def drain(buf, sem):
                pltpu.make_async_copy(pk_hbm.at[pl.ds(0, chunk)], buf[0], sem).wait()
                pltpu.make_async_copy(w_hbm.at[hd, pl.ds(0, chunk)], buf[1], sem).wait()

            def process(buf):
                pk_v, w_v = buf

                @plsc.parallel_loop(0, ngroups, unroll=_UNROLL)
                def _(g):
                    o = g * 16
                    pkv = pk_v[pl.ds(o, 16)]
                    sv = pkv & 0xFFFF
                    dv = lax.shift_right_logical(pkv, 16)
                    wv = w_v[pl.ds(o, 16)]
                    for j in range(npair):
                        pr = plsc.load_gather(tabs[j], [sv])
                        va, vb = plsc.unpack(plsc.bitcast(pr, jnp.bfloat16),
                                             format=plsc.PackFormat.INTERLEAVED,
                                             preferred_element_type=F32)
                        plsc.addupdate_scatter(accs[2 * j], [dv], va * wv)
                        plsc.addupdate_scatter(accs[2 * j + 1], [dv], vb * wv)

            for j in range(npair):
                pltpu.sync_copy(tab_hbm.at[ch0 // 2 + j], tabs[j].at[pl.ds(0, N)])

            def zero(i, cc):
                for k in range(nch):
                    accs[k][pl.ds(i * 16, 16)] = jnp.zeros((16,), F32)
                return cc
            lax.fori_loop(0, NP // 16, zero, 0)

            issue(0, bufA, semA)

            def do_pair(ci, cc):
                b = ci * (2 * chunk)
                issue(b + chunk, bufB, semB)
                drain(bufA, semA)
                process(bufA)

                @pl.when(ci + 1 < npairs)
                def _():
                    issue(b + 2 * chunk, bufA, semA)
                drain(bufB, semB)
                process(bufB)
                return cc
            lax.fori_loop(0, npairs, do_pair, 0)

            for k in range(nch):
                pltpu.sync_copy(accs[k].at[pl.ds(0, N)], out_t.at[ch0 + k])
            return c
        lax.fori_loop(0, npass, do_pass, 0)

    return body


def _msg_pass(tab, pk, w_e, npair, npass, nheads, chunk):
    nchan = 2 * tab.shape[0]
    body = _make_msg_body(npair, npass, nheads, chunk)
    ebuf = [pltpu.VMEM((chunk,), jnp.int32), pltpu.VMEM((chunk,), F32)]
    scr = ([pltpu.VMEM((NP,), jnp.int32)] * npair +
           [pltpu.VMEM((NP,), F32)] * (2 * npair) + ebuf + ebuf +
           [pltpu.SemaphoreType.DMA, pltpu.SemaphoreType.DMA])
    f = pl.kernel(
        body,
        out_type=jax.ShapeDtypeStruct((nchan, N), F32),
        mesh=_mesh(),
        scratch_types=scr,
        compiler_params=pltpu.CompilerParams(needs_layout_passes=False, use_tc_tiling_on_sc=False),
    )
    return f(tab, pk, w_e)


def _mid_body(acc_ref, sp_ref, bias_ref, w2_ref, ws2_ref, h2t_ref, a2t_ref):
    s8 = jnp.sum(sp_ref[...].reshape(HEADS, NW // HEADS, NP), axis=1)[:, :N]
    sbc = jnp.broadcast_to(s8[:, None, :], (HEADS, HID, N)).reshape(HEADS * HID, N)
    v = acc_ref[...] / (sbc + 1e-16) + bias_ref[...]
    h1n = jnp.where(v > 0, v, jnp.exp(v) - 1.0)
    h2t_ref[...] = lax.dot_general(w2_ref[...], h1n, (((0,), (0,)), ((), ())),
                                   preferred_element_type=F32)
    a2t_ref[...] = lax.dot_general(ws2_ref[...], h1n, (((0,), (0,)), ((), ())),
                                   preferred_element_type=F32)


def _final_body(acc2_ref, s2p_ref, bias2_ref, batch_ref, l1w_ref, l1b_ref,
                l2w_ref, l2b_ref, out_ref):
    s2 = jnp.sum(s2p_ref[...], axis=0, keepdims=True)[:, :N]
    v = acc2_ref[...] / (s2 + 1e-16) + bias2_ref[...]
    h = jnp.where(v > 0, v, jnp.exp(v) - 1.0)
    gid = lax.broadcasted_iota(jnp.int32, (N, NG), 1)
    P = (batch_ref[...] == gid).astype(F32)
    G = lax.dot_general(P, h, (((0,), (1,)), ((), ())),
                        preferred_element_type=F32)
    g1 = jnp.dot(G, l1w_ref[...], preferred_element_type=F32) + l1b_ref[...]
    g1 = jnp.where(g1 > 0, g1, jnp.exp(g1) - 1.0)
    lg = jnp.dot(g1, l2w_ref[...], preferred_element_type=F32) + l2b_ref[...]
    m = jnp.max(lg, axis=1, keepdims=True)
    lse = jnp.log(jnp.sum(jnp.exp(lg - m), axis=1, keepdims=True)) + m
    out_ref[...] = lg - lse


def kernel(x, edge_index, batch, W1, att_src1, att_dst1, bias1, W2, att_src2,
           att_dst2, bias2, lin1_W, lin1_b, lin2_W, lin2_b):
    sl = jnp.arange(N, dtype=jnp.int32)
    npad = E_PAD - E_REAL
    srcp = jnp.concatenate([edge_index[0].astype(jnp.int32), sl,
                            jnp.zeros((npad,), jnp.int32)])
    dstp = jnp.concatenate([edge_index[1].astype(jnp.int32), sl,
                            jnp.full((npad,), N, jnp.int32)])
    pk = srcp | (dstp << 16)

    w1r = W1.reshape(x.shape[1], HEADS, HID)
    ws1 = jnp.concatenate([jnp.einsum('khc,hc->kh', w1r, att_src1),
                           jnp.einsum('khc,hc->kh', w1r, att_dst1)], axis=1)
    ws2 = jnp.stack([W2 @ att_src2[0], W2 @ att_dst2[0]], axis=1)

    h1t, a1t = pl.pallas_call(
        _tc_in_body,
        out_shape=(jax.ShapeDtypeStruct((HEADS * HID, N), F32),
                   jax.ShapeDtypeStruct((2 * HEADS, N), F32)),
    )(x, W1, ws1)

    w1e, s1p = _attn_pass(a1t, pk, HEADS, 4096)
    out1t = _msg_pass(_pack_pairs(h1t), pk, w1e, npair=4, npass=2,
                      nheads=HEADS, chunk=2048)

    h2t, a2t = pl.pallas_call(
        _mid_body,
        out_shape=(jax.ShapeDtypeStruct((HID, N), F32),
                   jax.ShapeDtypeStruct((2, N), F32)),
        compiler_params=pltpu.CompilerParams(vmem_limit_bytes=100 * 2**20),
    )(out1t, s1p, bias1.reshape(HEADS * HID, 1), W2, ws2)

    w2e, s2p = _attn_pass(a2t, pk, 1, 2688)
    out2t = _msg_pass(_pack_pairs(h2t), pk, w2e, npair=1, npass=1, nheads=1,
                      chunk=4096)

    out = pl.pallas_call(
        _final_body,
        out_shape=jax.ShapeDtypeStruct((NG, 16), F32),
    )(out2t, s2p, bias2.reshape(HID, 1), batch.reshape(N, 1).astype(jnp.int32),
      lin1_W, lin1_b.reshape(1, HID), lin2_W, lin2_b.reshape(1, 16))
    return out


# speedup vs baseline: 41.7928x; 1.0001x over previous
import jax
import jax.numpy as jnp
from jax import lax
from jax.experimental import pallas as pl
from jax.experimental.pallas import tpu as pltpu
from jax.experimental.pallas import tpu_sc as plsc

N = 10000
NP = 10016
E_REAL = 330000
E_PAD = 344064
HEADS = 8
HID = 64
NG = 64
F32 = jnp.float32

NC, NS = 2, 16
NW = NC * NS


def _mesh():
    return plsc.VectorSubcoreMesh(core_axis_name="c", subcore_axis_name="s",
                                  num_cores=NC, num_subcores=NS)


def _tc_in_body(x_ref, w1_ref, ws1_ref, h1t_ref, a1t_ref):
    x = x_ref[...]
    h1t_ref[...] = lax.dot_general(w1_ref[...], x, (((0,), (1,)), ((), ())),
                                   preferred_element_type=F32)
    a1t_ref[...] = lax.dot_general(ws1_ref[...], x, (((0,), (1,)), ((), ())),
                                   preferred_element_type=F32)


_UNROLL = 8


def _make_attn_body(nheads, chunk):
    split = NW // nheads
    erange = E_PAD // split
    nchunks = erange // chunk
    ngroups = chunk // 16

    def body(a_hbm, pk_hbm, w_out, s_out, asrc_v, adst_v, s_v, pk_v, w_v):
        wid = lax.axis_index("s") * NC + lax.axis_index("c")
        hd = wid // split
        q = wid % split
        pltpu.sync_copy(a_hbm.at[hd], asrc_v.at[pl.ds(0, N)])
        pltpu.sync_copy(a_hbm.at[nheads + hd], adst_v.at[pl.ds(0, N)])
        adst_v[pl.ds(N, 16)] = jnp.full((16,), -30.0, F32)

        def zero(i, c):
            s_v[pl.ds(i * 16, 16)] = jnp.zeros((16,), F32)
            return c
        lax.fori_loop(0, NP // 16, zero, 0)

        base0 = q * erange

        def do_chunk(ci, c):
            b = base0 + ci * chunk
            pltpu.sync_copy(pk_hbm.at[pl.ds(b, chunk)], pk_v)

            @plsc.parallel_loop(0, ngroups, unroll=_UNROLL)
            def _(g):
                pkv = pk_v[pl.ds(g * 16, 16)]
                sv = pkv & 0xFFFF
                dv = lax.shift_right_logical(pkv, 16)
                e = plsc.load_gather(asrc_v, [sv]) + plsc.load_gather(adst_v, [dv])
                e = jnp.where(e > 0, e, 0.2 * e)
                w = jnp.exp(e)
                w_v[pl.ds(g * 16, 16)] = w
                plsc.addupdate_scatter(s_v, [dv], w)
            pltpu.sync_copy(w_v, w_out.at[hd, pl.ds(b, chunk)])
            return c
        lax.fori_loop(0, nchunks, do_chunk, 0)
        pltpu.sync_copy(s_v, s_out.at[wid])

    return body


def _attn_pass(a1t, pk, nheads, chunk):
    body = _make_attn_body(nheads, chunk)
    f = pl.kernel(
        body,
        out_type=(jax.ShapeDtypeStruct((nheads, E_PAD), F32),
                  jax.ShapeDtypeStruct((NW, NP), F32)),
        mesh=_mesh(),
        scratch_types=[
            pltpu.VMEM((NP,), F32), pltpu.VMEM((NP,), F32), pltpu.VMEM((NP,), F32),
            pltpu.VMEM((chunk,), jnp.int32), pltpu.VMEM((chunk,), F32),
        ],
        compiler_params=pltpu.CompilerParams(needs_layout_passes=False, use_tc_tiling_on_sc=False),
    )
    return f(a1t, pk)


def _pack_pairs(t):
    u = lax.bitcast_convert_type(t.astype(jnp.bfloat16), jnp.uint16).astype(jnp.uint32)
    return (u[0::2, :] | (u[1::2, :] << 16)).astype(jnp.int32)


def _make_msg_body(npair, npass, nheads, chunk):
    nch = 2 * npair
    ngroups = chunk // 16
    nchunks = E_PAD // chunk
    npairs = nchunks // 2
    assert nchunks % 2 == 0 and ngroups % _UNROLL == 0

    def body(tab_hbm, pk_hbm, w_hbm, out_t, *refs):
        tabs = refs[:npair]
        accs = refs[npair:npair + nch]
        bufA = refs[npair + nch:npair + nch + 2]
        bufB = refs[npair + nch + 2:npair + nch + 4]
        semA, semB = refs[npair + nch + 4:npair + nch + 6]
        wid = lax.axis_index("s") * NC + lax.axis_index("c")

        def do_pass(p, c):
            ch0 = p * (NW * nch) + wid * nch
            hd = ch0 // HID if nheads > 1 else 0

            def issue(b, buf, sem):
                pltpu.async_copy(pk_hbm.at[pl.ds(b, chunk)], buf[0], sem)
                pltpu.async_copy(w_hbm.at[hd, pl.ds(b, chunk)], buf[1], sem)

            def drain(buf, sem):
                pltpu.make_async_copy(pk_hbm.at[pl.ds(0, chunk)], buf[0], sem).wait()
                pltpu.make_async_copy(w_hbm.at[hd, pl.ds(0, chunk)], buf[1], sem).wait()

            def process(buf):
                pk_v, w_v = buf

                @plsc.parallel_loop(0, ngroups, unroll=_UNROLL)
                def _(g):
                    o = g * 16
                    pkv = pk_v[pl.ds(o, 16)]
                    sv = pkv & 0xFFFF
                    dv = lax.shift_right_logical(pkv, 16)
                    wv = w_v[pl.ds(o, 16)]
                    for j in range(npair):
                        pr = plsc.load_gather(tabs[j], [sv])
                        va, vb = plsc.unpack(plsc.bitcast(pr, jnp.bfloat16),
                                             format=plsc.PackFormat.INTERLEAVED,
                                             preferred_element_type=F32)
                        plsc.addupdate_scatter(accs[2 * j], [dv], va * wv)
                        plsc.addupdate_scatter(accs[2 * j + 1], [dv], vb * wv)

            for j in range(npair):
                pltpu.sync_copy(tab_hbm.at[ch0 // 2 + j], tabs[j].at[pl.ds(0, N)])

            def zero(i, cc):
                for k in range(nch):
                    accs[k][pl.ds(i * 16, 16)] = jnp.zeros((16,), F32)
                return cc
            lax.fori_loop(0, NP // 16, zero, 0)

            issue(0, bufA, semA)

            def do_pair(ci, cc):
                b = ci * (2 * chunk)
                issue(b + chunk, bufB, semB)
                drain(bufA, semA)
                process(bufA)

                @pl.when(ci + 1 < npairs)
                def _():
                    issue(b + 2 * chunk, bufA, semA)
                drain(bufB, semB)
                process(bufB)
                return cc
            lax.fori_loop(0, npairs, do_pair, 0)

            for k in range(nch):
                pltpu.sync_copy(accs[k].at[pl.ds(0, N)], out_t.at[ch0 + k])
            return c
        lax.fori_loop(0, npass, do_pass, 0)

    return body


def _msg_pass(tab, pk, w_e, npair, npass, nheads, chunk):
    nchan = 2 * tab.shape[0]
    body = _make_msg_body(npair, npass, nheads, chunk)
    ebuf = [pltpu.VMEM((chunk,), jnp.int32), pltpu.VMEM((chunk,), F32)]
    scr = ([pltpu.VMEM((NP,), jnp.int32)] * npair +
           [pltpu.VMEM((NP,), F32)] * (2 * npair) + ebuf + ebuf +
           [pltpu.SemaphoreType.DMA, pltpu.SemaphoreType.DMA])
    f = pl.kernel(
        body,
        out_type=jax.ShapeDtypeStruct((nchan, N), F32),
        mesh=_mesh(),
        scratch_types=scr,
        compiler_params=pltpu.CompilerParams(needs_layout_passes=False, use_tc_tiling_on_sc=False),
    )
    return f(tab, pk, w_e)


def _mid_body(acc_ref, sp_ref, bias_ref, w2_ref, ws2_ref, h2t_ref, a2t_ref):
    s8 = jnp.sum(sp_ref[...].reshape(HEADS, NW // HEADS, NP), axis=1)[:, :N]
    sbc = jnp.broadcast_to(s8[:, None, :], (HEADS, HID, N)).reshape(HEADS * HID, N)
    v = acc_ref[...] / (sbc + 1e-16) + bias_ref[...]
    h1n = jnp.where(v > 0, v, jnp.exp(v) - 1.0)
    h2t_ref[...] = lax.dot_general(w2_ref[...], h1n, (((0,), (0,)), ((), ())),
                                   preferred_element_type=F32)
    a2t_ref[...] = lax.dot_general(ws2_ref[...], h1n, (((0,), (0,)), ((), ())),
                                   preferred_element_type=F32)


def _final_body(acc2_ref, s2p_ref, bias2_ref, batch_ref, l1w_ref, l1b_ref,
                l2w_ref, l2b_ref, out_ref):
    s2 = jnp.sum(s2p_ref[...], axis=0, keepdims=True)[:, :N]
    v = acc2_ref[...] / (s2 + 1e-16) + bias2_ref[...]
    h = jnp.where(v > 0, v, jnp.exp(v) - 1.0)
    gid = lax.broadcasted_iota(jnp.int32, (N, NG), 1)
    P = (batch_ref[...] == gid).astype(F32)
    G = lax.dot_general(P, h, (((0,), (1,)), ((), ())),
                        preferred_element_type=F32)
    g1 = jnp.dot(G, l1w_ref[...], preferred_element_type=F32) + l1b_ref[...]
    g1 = jnp.where(g1 > 0, g1, jnp.exp(g1) - 1.0)
    lg = jnp.dot(g1, l2w_ref[...], preferred_element_type=F32) + l2b_ref[...]
    m = jnp.max(lg, axis=1, keepdims=True)
    lse = jnp.log(jnp.sum(jnp.exp(lg - m), axis=1, keepdims=True)) + m
    out_ref[...] = lg - lse


def kernel(x, edge_index, batch, W1, att_src1, att_dst1, bias1, W2, att_src2,
           att_dst2, bias2, lin1_W, lin1_b, lin2_W, lin2_b):
    sl = jnp.arange(N, dtype=jnp.int32)
    npad = E_PAD - E_REAL
    srcp = jnp.concatenate([edge_index[0].astype(jnp.int32), sl,
                            jnp.zeros((npad,), jnp.int32)])
    dstp = jnp.concatenate([edge_index[1].astype(jnp.int32), sl,
                            jnp.full((npad,), N, jnp.int32)])
    pk = srcp | (dstp << 16)

    w1r = W1.reshape(x.shape[1], HEADS, HID)
    ws1 = jnp.concatenate([jnp.einsum('khc,hc->kh', w1r, att_src1),
                           jnp.einsum('khc,hc->kh', w1r, att_dst1)], axis=1)
    ws2 = jnp.stack([W2 @ att_src2[0], W2 @ att_dst2[0]], axis=1)

    h1t, a1t = pl.pallas_call(
        _tc_in_body,
        out_shape=(jax.ShapeDtypeStruct((HEADS * HID, N), F32),
                   jax.ShapeDtypeStruct((2 * HEADS, N), F32)),
    )(x, W1, ws1)

    w1e, s1p = _attn_pass(a1t, pk, HEADS, 4096)
    out1t = _msg_pass(_pack_pairs(h1t), pk, w1e, npair=4, npass=2,
                      nheads=HEADS, chunk=2048)

    h2t, a2t = pl.pallas_call(
        _mid_body,
        out_shape=(jax.ShapeDtypeStruct((HID, N), F32),
                   jax.ShapeDtypeStruct((2, N), F32)),
        compiler_params=pltpu.CompilerParams(vmem_limit_bytes=100 * 2**20),
    )(out1t, s1p, bias1.reshape(HEADS * HID, 1), W2, ws2)

    w2e, s2p = _attn_pass(a2t, pk, 1, 2688)
    out2t = _msg_pass(_pack_pairs(h2t), pk, w2e, npair=1, npass=1, nheads=1,
                      chunk=4096)

    out = pl.pallas_call(
        _final_body,
        out_shape=jax.ShapeDtypeStruct((NG, 16), F32),
    )(out2t, s2p, bias2.reshape(HID, 1), batch.reshape(N, 1).astype(jnp.int32),
      lin1_W, lin1_b.reshape(1, HID), lin2_W, lin2_b.reshape(1, 16))
    return out
